# baseline (speedup 1.0000x reference)
import functools

import jax
import jax.numpy as jnp
from jax import lax
from jax.experimental import pallas as pl
from jax.experimental.pallas import tpu as pltpu
from jax.experimental.pallas import tpu_sc as plsc

D = 32
T = 50
NTOK = 16384
NC, NS = 2, 16
NW = NC * NS
CHUNK = 128
NU = T * (NTOK // CHUNK)
UPW = NU // NW
NBUF = 4
NT = 2
NGRP = UPW // NBUF


def _make_kernel():
    mesh = plsc.VectorSubcoreMesh(core_axis_name="c", subcore_axis_name="s")

    @functools.partial(
        pl.kernel,
        mesh=mesh,
        out_type=jax.ShapeDtypeStruct((T, D // 8, NTOK // CHUNK, 8, CHUNK),
                                      jnp.float32),
        scratch_types=[
            pltpu.VMEM((UPW, CHUNK), jnp.int32),
            pltpu.VMEM((NBUF, CHUNK, 128), jnp.float32),
            pltpu.VMEM((NT, D // 8, 8, CHUNK), jnp.float32),
            pltpu.SemaphoreType.DMA((NBUF,)),
            pltpu.SemaphoreType.DMA((NT,)),
        ],
        compiler_params=pltpu.CompilerParams(use_tc_tiling_on_sc=True,
                                             needs_layout_passes=False),
    )
    def emb_kernel(idx_hbm, table_hbm, out_hbm, idx_v, rows_v, trans_v,
                   gsem, osem):
        wid = lax.axis_index("s") * NC + lax.axis_index("c")
        u0 = wid * UPW
        pltpu.sync_copy(idx_hbm.at[pl.ds(u0, UPW)], idx_v)

        iota = lax.iota(jnp.int32, 16)

        def start_gather(k, b):
            pltpu.async_copy(table_hbm.at[idx_v.at[k]], rows_v.at[b],
                             gsem.at[b])

        def wait_gather(k, b):
            pltpu.make_async_copy(table_hbm.at[idx_v.at[k]], rows_v.at[b],
                                  gsem.at[b]).wait()

        def out_slice(k):
            u = u0 + k
            j = u // (NTOK // CHUNK)
            tb = u % (NTOK // CHUNK)
            return out_hbm.at[j, :, tb]

        def transpose(b, t):
            src = rows_v.at[b]
            for blk in range(CHUNK // 16):
                tc = iota + (blk * 16)
                vecs = [plsc.load_gather(src, [tc, jnp.full((16,), f, jnp.int32)])
                        for f in range(D)]
                for f in range(D):
                    trans_v[t, f // 8, f % 8, pl.ds(blk * 16, 16)] = vecs[f]

        def start_out(k, t):
            pltpu.async_copy(trans_v.at[t], out_slice(k), osem.at[t])

        def wait_out(k, t):
            pltpu.make_async_copy(trans_v.at[t], out_slice(k),
                                  osem.at[t]).wait()

        for b in range(NBUF):
            start_gather(b, b)

        def step(k, b, lookahead):
            t = b % NT
            wait_gather(k, b)

            @pl.when(k >= NT)
            def _():
                wait_out(k, t)

            transpose(b, t)
            start_out(k, t)
            if lookahead:
                start_gather(k + NBUF, b)

        def grp_body(g, carry):
            for b in range(NBUF):
                step(g * NBUF + b, b, True)
            return carry

        lax.fori_loop(0, NGRP - 1, grp_body, 0)

        for b in range(NBUF):
            step((NGRP - 1) * NBUF + b, b, False)

        for b in range(NBUF - NT, NBUF):
            k = (NGRP - 1) * NBUF + b
            wait_out(k, b % NT)

    return emb_kernel


_emb = _make_kernel()


@jax.jit
def kernel(token_ids, weights):
    idx = token_ids.T.reshape(NU, CHUNK)
    wpad = jnp.pad(weights, ((0, 0), (0, 128 - D)))
    out5d = _emb(idx, wpad)
    return out5d.transpose(2, 4, 0, 1, 3).reshape(NTOK, T, D)


# speedup vs baseline: 1.0005x; 1.0005x over previous
import functools

import jax
import jax.numpy as jnp
from jax import lax
from jax.experimental import pallas as pl
from jax.experimental.pallas import tpu as pltpu
from jax.experimental.pallas import tpu_sc as plsc

D = 32
T = 50
NTOK = 16384
NC, NS = 2, 16
NW = NC * NS
CHUNK = 128
NU = T * (NTOK // CHUNK)
UPW = NU // NW
NBUF = 4
NT = 4
NGRP = UPW // NBUF


def _make_kernel():
    mesh = plsc.VectorSubcoreMesh(core_axis_name="c", subcore_axis_name="s")

    @functools.partial(
        pl.kernel,
        mesh=mesh,
        out_type=jax.ShapeDtypeStruct((T, D // 8, NTOK // CHUNK, 8, CHUNK),
                                      jnp.float32),
        scratch_types=[
            pltpu.VMEM((UPW, CHUNK), jnp.int32),
            pltpu.VMEM((NBUF, CHUNK, 128), jnp.float32),
            pltpu.VMEM((NT, D // 8, 8, CHUNK), jnp.float32),
            pltpu.SemaphoreType.DMA((NBUF,)),
            pltpu.SemaphoreType.DMA((NT,)),
        ],
        compiler_params=pltpu.CompilerParams(use_tc_tiling_on_sc=True,
                                             needs_layout_passes=False),
    )
    def emb_kernel(idx_hbm, table_hbm, out_hbm, idx_v, rows_v, trans_v,
                   gsem, osem):
        wid = lax.axis_index("s") * NC + lax.axis_index("c")
        u0 = wid * UPW
        pltpu.sync_copy(idx_hbm.at[pl.ds(u0, UPW)], idx_v)

        iota = lax.iota(jnp.int32, 16)

        def start_gather(k, b):
            pltpu.async_copy(table_hbm.at[idx_v.at[k]], rows_v.at[b],
                             gsem.at[b])

        def wait_gather(k, b):
            pltpu.make_async_copy(table_hbm.at[idx_v.at[k]], rows_v.at[b],
                                  gsem.at[b]).wait()

        def out_slice(k):
            u = u0 + k
            j = u // (NTOK // CHUNK)
            tb = u % (NTOK // CHUNK)
            return out_hbm.at[j, :, tb]

        def transpose(b, t):
            src = rows_v.at[b]
            for blk in range(CHUNK // 16):
                tc = iota + (blk * 16)
                vecs = [plsc.load_gather(src, [tc, jnp.full((16,), f, jnp.int32)])
                        for f in range(D)]
                for f in range(D):
                    trans_v[t, f // 8, f % 8, pl.ds(blk * 16, 16)] = vecs[f]

        def start_out(k, t):
            pltpu.async_copy(trans_v.at[t], out_slice(k), osem.at[t])

        def wait_out(k, t):
            pltpu.make_async_copy(trans_v.at[t], out_slice(k),
                                  osem.at[t]).wait()

        for b in range(NBUF):
            start_gather(b, b)

        def step(k, b, lookahead):
            t = b % NT
            wait_gather(k, b)

            @pl.when(k >= NT)
            def _():
                wait_out(k, t)

            transpose(b, t)
            start_out(k, t)
            if lookahead:
                start_gather(k + NBUF, b)

        def grp_body(g, carry):
            for b in range(NBUF):
                step(g * NBUF + b, b, True)
            return carry

        lax.fori_loop(0, NGRP - 1, grp_body, 0)

        for b in range(NBUF):
            step((NGRP - 1) * NBUF + b, b, False)

        for b in range(NBUF - NT, NBUF):
            k = (NGRP - 1) * NBUF + b
            wait_out(k, b % NT)

    return emb_kernel


_emb = _make_kernel()


@jax.jit
def kernel(token_ids, weights):
    idx = token_ids.T.reshape(NU, CHUNK)
    wpad = jnp.pad(weights, ((0, 0), (0, 128 - D)))
    out5d = _emb(idx, wpad)
    return out5d.transpose(2, 4, 0, 1, 3).reshape(NTOK, T, D)
